# Initial kernel scaffold; baseline (speedup 1.0000x reference)
import jax
import jax.numpy as jnp
from jax import lax
from jax.experimental import pallas as pl
from jax.experimental.pallas import tpu as pltpu
from jax.experimental.pallas import tpu_sc as plsc

SEQ = 200
DIM = 32
BATCH = 4096
B_TOTAL = BATCH * SEQ
NC, NS = 2, 16
NW = NC * NS
B_PER_W = B_TOTAL // NW
GATHER = 128
CHUNK = 512
GPC = CHUNK // GATHER
N_CHUNK = B_PER_W // CHUNK
IDX_ROWS = B_PER_W // GATHER
POS_EXT = 4 * SEQ
LANES = 16
NBUF = 2
UNROLL = 8


def _body(idx_hbm, word_hbm, pos_hbm, out_hbm,
          idx_v, pos_v, rows_in, rows_out, gsem0, gsem1):
    cid = lax.axis_index("c")
    sid = lax.axis_index("s")
    wid = sid * NC + cid
    base_row = wid * B_PER_W
    idx_row0 = wid * IDX_ROWS

    pltpu.sync_copy(idx_hbm.at[pl.ds(idx_row0, IDX_ROWS)], idx_v)
    pltpu.sync_copy(pos_hbm, pos_v)

    gsems = (gsem0, gsem1)

    def issue(g, buf):
        for i in range(GPC):
            pltpu.async_copy(
                word_hbm.at[idx_v.at[g * GPC + i]],
                rows_in.at[buf, pl.ds(i * GATHER, GATHER)],
                gsems[buf])

    def drain(g, buf):
        for i in range(GPC):
            pltpu.make_async_copy(
                word_hbm.at[idx_v.at[g * GPC + i]],
                rows_in.at[buf, pl.ds(i * GATHER, GATHER)],
                gsems[buf]).wait()

    issue(0, 0)

    @pl.loop(0, N_CHUNK, step=NBUF)
    def _chunks(g0):
        for db in range(NBUF):
            g = g0 + db
            buf = db

            @pl.when(g + 1 < N_CHUNK)
            def _():
                issue(g + 1, 1 - db)

            drain(g, buf)

            r = lax.rem(g * CHUNK, SEQ)

            @pl.loop(0, CHUNK, step=UNROLL)
            def _rows(j):
                for u in range(UNROLL):
                    row = j + u
                    for h in range(DIM // LANES):
                        sl = pl.ds(h * LANES, LANES)
                        rows_out[buf, row, sl] = (
                            rows_in[buf, row, sl] + pos_v[r + row, sl])

            pltpu.sync_copy(
                rows_out.at[buf],
                out_hbm.at[pl.ds(base_row + g * CHUNK, CHUNK)])


_mesh = plsc.VectorSubcoreMesh(core_axis_name="c", subcore_axis_name="s")

_sc_call = pl.kernel(
    _body,
    out_type=jax.ShapeDtypeStruct((B_TOTAL, DIM), jnp.float32),
    mesh=_mesh,
    scratch_types=[
        pltpu.VMEM((IDX_ROWS, GATHER), jnp.int32),
        pltpu.VMEM((POS_EXT, DIM), jnp.float32),
        pltpu.VMEM((NBUF, CHUNK, DIM), jnp.float32),
        pltpu.VMEM((NBUF, CHUNK, DIM), jnp.float32),
        pltpu.SemaphoreType.DMA,
        pltpu.SemaphoreType.DMA,
    ],
)


@jax.jit
def kernel(inputs, word_table, pos_table):
    idx = inputs.reshape(-1).astype(jnp.int32).reshape(B_TOTAL // GATHER,
                                                       GATHER)
    pos_ext = jnp.concatenate([pos_table] * (POS_EXT // SEQ), axis=0)
    out = _sc_call(idx, word_table, pos_ext)
    return out.reshape(BATCH, SEQ, DIM)


# speedup vs baseline: 1.2147x; 1.2147x over previous
import jax
import jax.numpy as jnp
from jax import lax
from jax.experimental import pallas as pl
from jax.experimental.pallas import tpu as pltpu
from jax.experimental.pallas import tpu_sc as plsc

SEQ = 200
DIM = 32
BATCH = 4096
B_TOTAL = BATCH * SEQ
NC, NS = 2, 16
NW = NC * NS
B_PER_W = B_TOTAL // NW
GATHER = 128
CHUNK = 512
GPC = CHUNK // GATHER
N_CHUNK = B_PER_W // CHUNK
IDX_ROWS = B_PER_W // GATHER
POS_EXT = 4 * SEQ
LANES = 16
NBUF = 2
UNROLL = 8


def _body(idx_hbm, word_hbm, pos_hbm, out_hbm,
          idx_v, pos_v, rows_in, rows_out, gsem0, gsem1):
    cid = lax.axis_index("c")
    sid = lax.axis_index("s")
    wid = sid * NC + cid
    base_row = wid * B_PER_W
    idx_row0 = wid * IDX_ROWS

    pltpu.sync_copy(idx_hbm.at[pl.ds(idx_row0, IDX_ROWS)], idx_v)
    pltpu.sync_copy(pos_hbm, pos_v)

    gsems = (gsem0, gsem1)

    def issue(g, buf):
        for i in range(GPC):
            pltpu.async_copy(
                word_hbm.at[idx_v.at[g * GPC + i]],
                rows_in.at[buf, pl.ds(i * GATHER, GATHER)],
                gsems[buf])

    def drain(g, buf):
        for i in range(GPC):
            pltpu.make_async_copy(
                word_hbm.at[idx_v.at[g * GPC + i]],
                rows_in.at[buf, pl.ds(i * GATHER, GATHER)],
                gsems[buf]).wait()

    issue(0, 0)

    @pl.loop(0, N_CHUNK, step=NBUF)
    def _chunks(g0):
        for db in range(NBUF):
            g = g0 + db
            buf = db

            @pl.when(g + 1 < N_CHUNK)
            def _():
                issue(g + 1, 1 - db)

            drain(g, buf)

            r = lax.rem(g * CHUNK, SEQ)

            @pl.loop(0, CHUNK, step=UNROLL)
            def _rows(j):
                for u in range(UNROLL):
                    row = j + u
                    for h in range(DIM // LANES):
                        sl = pl.ds(h * LANES, LANES)
                        rows_out[buf, row, sl] = (
                            rows_in[buf, row, sl] + pos_v[r + row, sl])

            pltpu.sync_copy(
                rows_out.at[buf],
                out_hbm.at[pl.ds(base_row + g * CHUNK, CHUNK)])


_mesh = plsc.VectorSubcoreMesh(core_axis_name="c", subcore_axis_name="s")

_sc_call = pl.kernel(
    _body,
    out_type=jax.ShapeDtypeStruct((B_TOTAL, DIM), jnp.float32),
    mesh=_mesh,
    scratch_types=[
        pltpu.VMEM((IDX_ROWS, GATHER), jnp.int32),
        pltpu.VMEM((POS_EXT, DIM), jnp.float32),
        pltpu.VMEM((NBUF, CHUNK, DIM), jnp.float32),
        pltpu.VMEM((NBUF, CHUNK, DIM), jnp.float32),
        pltpu.SemaphoreType.DMA,
        pltpu.SemaphoreType.DMA,
    ],
    compiler_params=pltpu.CompilerParams(use_tc_tiling_on_sc=False),
)


@jax.jit
def kernel(inputs, word_table, pos_table):
    idx = inputs.reshape(-1).astype(jnp.int32).reshape(B_TOTAL // GATHER,
                                                       GATHER)
    pos_ext = jnp.concatenate([pos_table] * (POS_EXT // SEQ), axis=0)
    out = _sc_call(idx, word_table, pos_ext)
    return out.reshape(BATCH, SEQ, DIM)
